# baseline (speedup 1.0000x reference)
import functools
import math

import jax
import jax.numpy as jnp
from jax.experimental import pallas as pl
from jax.experimental.pallas import tpu as pltpu

_IN_F = 4096
_OUT_F = 64
_K = 8
_NEG_INF = float("-inf")


def _fused_body(x_ref, wt_ref, b_ref, vals_ref, idx_ref):
    x = x_ref[...]
    wt = wt_ref[...]
    logits = jax.lax.dot_general(
        x, wt, (((1,), (0,)), ((), ())),
        preferred_element_type=jnp.float32,
    ) + b_ref[...]

    col = jax.lax.broadcasted_iota(jnp.int32, logits.shape, 1)
    vals_cols = []
    idx_cols = []
    for _ in range(_K):
        m = jnp.max(logits, axis=1, keepdims=True)
        pick = jnp.min(
            jnp.where(logits == m, col, _OUT_F), axis=1, keepdims=True
        )
        vals_cols.append(m)
        idx_cols.append(pick)
        logits = jnp.where(col == pick, _NEG_INF, logits)
    vals_ref[...] = jnp.concatenate(vals_cols, axis=1)
    idx_ref[...] = jnp.concatenate(idx_cols, axis=1)


@functools.partial(jax.jit, static_argnames=("block",))
def _run(x, wt, b2d, block=1024):
    n = x.shape[0]
    grid = (n // block,)
    return pl.pallas_call(
        _fused_body,
        grid=grid,
        in_specs=[
            pl.BlockSpec((block, _IN_F), lambda i: (i, 0)),
            pl.BlockSpec((_IN_F, _OUT_F), lambda i: (0, 0)),
            pl.BlockSpec((1, _OUT_F), lambda i: (0, 0)),
        ],
        out_specs=[
            pl.BlockSpec((block, _K), lambda i: (i, 0)),
            pl.BlockSpec((block, _K), lambda i: (i, 0)),
        ],
        out_shape=[
            jax.ShapeDtypeStruct((n, _K), jnp.float32),
            jax.ShapeDtypeStruct((n, _K), jnp.int32),
        ],
        compiler_params=pltpu.CompilerParams(
            dimension_semantics=("parallel",),
        ),
    )(x, wt, b2d)


def kernel(input, weight, bias):
    wt = weight.T
    b2d = bias.reshape(1, _OUT_F)
    vals, idx = _run(input, wt, b2d)
    return (vals, idx)


# speedup vs baseline: 1.0004x; 1.0004x over previous
import functools
import math

import jax
import jax.numpy as jnp
from jax.experimental import pallas as pl
from jax.experimental.pallas import tpu as pltpu

_IN_F = 4096
_OUT_F = 64
_K = 8
_NEG_INF = float("-inf")


def _fused_body(x_ref, wt_ref, b_ref, vals_ref, idx_ref):
    x = x_ref[...]
    wt = wt_ref[...]
    logits = jax.lax.dot_general(
        x, wt, (((1,), (0,)), ((), ())),
        preferred_element_type=jnp.float32,
    ) + b_ref[...]

    col = jax.lax.broadcasted_iota(jnp.int32, logits.shape, 1)
    vals_cols = []
    idx_cols = []
    for _ in range(_K):
        m = jnp.max(logits, axis=1, keepdims=True)
        pick = jnp.min(
            jnp.where(logits == m, col, _OUT_F), axis=1, keepdims=True
        )
        vals_cols.append(m)
        idx_cols.append(pick)
        logits = jnp.where(col == pick, _NEG_INF, logits)
    vals_ref[...] = jnp.concatenate(vals_cols, axis=1)
    idx_ref[...] = jnp.concatenate(idx_cols, axis=1)


@functools.partial(jax.jit, static_argnames=("block",))
def _run(x, wt, b2d, block=1024):
    n = x.shape[0]
    grid = (n // block,)
    return pl.pallas_call(
        _fused_body,
        grid=grid,
        in_specs=[
            pl.BlockSpec((block, _IN_F), lambda i: (i, 0)),
            pl.BlockSpec((_IN_F, _OUT_F), lambda i: (0, 0)),
            pl.BlockSpec((1, _OUT_F), lambda i: (0, 0)),
        ],
        out_specs=[
            pl.BlockSpec((block, _K), lambda i: (i, 0)),
            pl.BlockSpec((block, _K), lambda i: (i, 0)),
        ],
        out_shape=[
            jax.ShapeDtypeStruct((n, _K), jnp.float32),
            jax.ShapeDtypeStruct((n, _K), jnp.int32),
        ],
        compiler_params=pltpu.CompilerParams(
            dimension_semantics=("arbitrary",),
        ),
    )(x, wt, b2d)


def kernel(input, weight, bias):
    wt = weight.T
    b2d = bias.reshape(1, _OUT_F)
    vals, idx = _run(input, wt, b2d)
    return (vals, idx)
